# baseline (speedup 1.0000x reference)
import functools

import jax
import jax.numpy as jnp
from jax import lax
from jax.experimental import pallas as pl
from jax.experimental.pallas import tpu as pltpu
from jax.experimental.pallas import tpu_sc as plsc

_NUM_BRACKETS = 20
_EMBED_DIM = 32
_ELO_MIN = 800.0
_ELO_MAX = 2800.0
_BRACKET_SIZE = (_ELO_MAX - _ELO_MIN) / _NUM_BRACKETS
_LANES = 16
_NC = 2
_NS = 16
_NW = _NC * _NS


@functools.lru_cache(maxsize=None)
def _build(batch: int):
    bpw = batch // _NW
    ngroups = bpw // _LANES
    mesh = plsc.VectorSubcoreMesh(core_axis_name="c", subcore_axis_name="s")

    def body(elo_hbm, table_hbm, out_hbm, elo_v, tabt_v, out_v):
        wid = lax.axis_index("s") * _NC + lax.axis_index("c")
        base = wid * bpw
        pltpu.sync_copy(table_hbm, tabt_v)
        pltpu.sync_copy(elo_hbm.at[pl.ds(base, bpw)], elo_v)

        def group(g, carry):
            eg = elo_v[pl.ds(g * _LANES, _LANES)]
            ef = jnp.clip(eg.astype(jnp.float32), _ELO_MIN, _ELO_MAX - 1.0)
            bf = (ef - _ELO_MIN) / _BRACKET_SIZE
            lo = bf.astype(jnp.int32)
            up = jnp.minimum(lo + 1, _NUM_BRACKETS - 1)
            alpha = bf - lo.astype(jnp.float32)
            col = g * _LANES

            @plsc.parallel_loop(0, _EMBED_DIM, step=1, unroll=8)
            def dim_loop(d):
                dsplat = jnp.full((_LANES,), 1, jnp.int32) * d
                t = plsc.load_gather(tabt_v, [dsplat, lo])
                u = plsc.load_gather(tabt_v, [dsplat, up])
                out_v[d, pl.ds(col, _LANES)] = t + alpha * (u - t)
            return carry

        lax.fori_loop(0, ngroups, group, 0)

        pltpu.sync_copy(out_v, out_hbm.at[:, pl.ds(base, bpw)])

    return pl.kernel(
        body,
        out_type=jax.ShapeDtypeStruct((_EMBED_DIM, batch), jnp.float32),
        mesh=mesh,
        compiler_params=pltpu.CompilerParams(
            needs_layout_passes=False, skip_device_barrier=True
        ),
        scratch_types=[
            pltpu.VMEM((bpw,), jnp.int32),
            pltpu.VMEM((_EMBED_DIM, _NUM_BRACKETS), jnp.float32),
            pltpu.VMEM((_EMBED_DIM, bpw), jnp.float32),
        ],
    )


def kernel(elo, table):
    out_t = _build(elo.shape[0])(elo, table.T)
    return out_t.T


# speedup vs baseline: 1.0002x; 1.0002x over previous
import functools

import jax
import jax.numpy as jnp
from jax import lax
from jax.experimental import pallas as pl
from jax.experimental.pallas import tpu as pltpu
from jax.experimental.pallas import tpu_sc as plsc

_NUM_BRACKETS = 20
_EMBED_DIM = 32
_ELO_MIN = 800.0
_ELO_MAX = 2800.0
_BRACKET_SIZE = (_ELO_MAX - _ELO_MIN) / _NUM_BRACKETS
_LANES = 16
_NC = 2
_NS = 16
_NW = _NC * _NS


@functools.lru_cache(maxsize=None)
def _build(batch: int):
    bpw = batch // _NW
    ngroups = bpw // _LANES
    mesh = plsc.VectorSubcoreMesh(core_axis_name="c", subcore_axis_name="s")

    def body(elo_hbm, table_hbm, out_hbm, elo_v, tabt_v, out_v):
        wid = lax.axis_index("s") * _NC + lax.axis_index("c")
        base = wid * bpw
        pltpu.sync_copy(table_hbm, tabt_v)
        pltpu.sync_copy(elo_hbm.at[pl.ds(base, bpw)], elo_v)

        @plsc.parallel_loop(0, ngroups, step=1, unroll=1)
        def group(g):
            eg = elo_v[pl.ds(g * _LANES, _LANES)]
            ef = jnp.clip(eg.astype(jnp.float32), _ELO_MIN, _ELO_MAX - 1.0)
            bf = (ef - _ELO_MIN) / _BRACKET_SIZE
            lo = bf.astype(jnp.int32)
            up = jnp.minimum(lo + 1, _NUM_BRACKETS - 1)
            alpha = bf - lo.astype(jnp.float32)
            col = g * _LANES

            @plsc.parallel_loop(0, _EMBED_DIM, step=1, unroll=4)
            def dim_loop(d):
                dsplat = jnp.full((_LANES,), 1, jnp.int32) * d
                t = plsc.load_gather(tabt_v, [dsplat, lo])
                u = plsc.load_gather(tabt_v, [dsplat, up])
                out_v[d, pl.ds(col, _LANES)] = t + alpha * (u - t)

        pltpu.sync_copy(out_v, out_hbm.at[:, pl.ds(base, bpw)])

    return pl.kernel(
        body,
        out_type=jax.ShapeDtypeStruct((_EMBED_DIM, batch), jnp.float32),
        mesh=mesh,
        compiler_params=pltpu.CompilerParams(
            needs_layout_passes=False, skip_device_barrier=True
        ),
        scratch_types=[
            pltpu.VMEM((bpw,), jnp.int32),
            pltpu.VMEM((_EMBED_DIM, _NUM_BRACKETS), jnp.float32),
            pltpu.VMEM((_EMBED_DIM, bpw), jnp.float32),
        ],
    )


def kernel(elo, table):
    out_t = _build(elo.shape[0])(elo, table.T)
    return out_t.T


# speedup vs baseline: 1.0092x; 1.0089x over previous
import functools

import jax
import jax.numpy as jnp
from jax import lax
from jax.experimental import pallas as pl
from jax.experimental.pallas import tpu as pltpu
from jax.experimental.pallas import tpu_sc as plsc

_NUM_BRACKETS = 20
_EMBED_DIM = 32
_ELO_MIN = 800.0
_ELO_MAX = 2800.0
_BRACKET_SIZE = (_ELO_MAX - _ELO_MIN) / _NUM_BRACKETS
_LANES = 16
_NC = 2
_NS = 16
_NW = _NC * _NS


@functools.lru_cache(maxsize=None)
def _build(batch: int):
    bpw = batch // _NW
    ngroups = bpw // _LANES
    mesh = plsc.VectorSubcoreMesh(core_axis_name="c", subcore_axis_name="s")

    def body(elo_hbm, table_hbm, out_hbm, elo_v, tabt_v, out_v, sem):
        wid = lax.axis_index("s") * _NC + lax.axis_index("c")
        base = wid * bpw
        half = bpw // 2
        pltpu.sync_copy(table_hbm, tabt_v)
        pltpu.sync_copy(elo_hbm.at[pl.ds(base, bpw)], elo_v)

        def group_body(g):
            eg = elo_v[pl.ds(g * _LANES, _LANES)]
            ef = jnp.clip(eg.astype(jnp.float32), _ELO_MIN, _ELO_MAX - 1.0)
            bf = (ef - _ELO_MIN) / _BRACKET_SIZE
            lo = bf.astype(jnp.int32)
            up = jnp.minimum(lo + 1, _NUM_BRACKETS - 1)
            alpha = bf - lo.astype(jnp.float32)
            col = g * _LANES

            @plsc.parallel_loop(0, _EMBED_DIM, step=1, unroll=4)
            def dim_loop(d):
                dsplat = jnp.full((_LANES,), 1, jnp.int32) * d
                t = plsc.load_gather(tabt_v, [dsplat, lo])
                u = plsc.load_gather(tabt_v, [dsplat, up])
                out_v[d, pl.ds(col, _LANES)] = t + alpha * (u - t)

        plsc.parallel_loop(0, ngroups // 2, step=1, unroll=1)(group_body)
        first = pltpu.async_copy(
            out_v.at[:, pl.ds(0, half)], out_hbm.at[:, pl.ds(base, half)], sem
        )
        plsc.parallel_loop(ngroups // 2, ngroups, step=1, unroll=1)(group_body)
        pltpu.sync_copy(
            out_v.at[:, pl.ds(half, half)],
            out_hbm.at[:, pl.ds(base + half, half)],
        )
        first.wait()

    return pl.kernel(
        body,
        out_type=jax.ShapeDtypeStruct((_EMBED_DIM, batch), jnp.float32),
        mesh=mesh,
        compiler_params=pltpu.CompilerParams(
            needs_layout_passes=False, skip_device_barrier=True
        ),
        scratch_types=[
            pltpu.VMEM((bpw,), jnp.int32),
            pltpu.VMEM((_EMBED_DIM, _NUM_BRACKETS), jnp.float32),
            pltpu.VMEM((_EMBED_DIM, bpw), jnp.float32),
            pltpu.SemaphoreType.DMA,
        ],
    )


def kernel(elo, table):
    out_t = _build(elo.shape[0])(elo, table.T)
    return out_t.T
